# baseline (speedup 1.0000x reference)
import numpy as np
import jax
import jax.numpy as jnp
from jax import lax
from jax.experimental import pallas as pl
from jax.experimental.pallas import tpu as pltpu
from jax.experimental.pallas import tpu_sc as plsc

_D = 31
_DD = _D * _D
_DDP = 976
_ROW_BLOCKS = 34
_IMG_BLOCKS = 542 * _ROW_BLOCKS
_NJOBS = 8
_NW = 32
_M = 8
_UNITS = 4 * 2048 // (_NW * _M)
_NBLK = 96
_NP_ = 2 * _M
_ZROW = 93 * 16

_j = np.arange(_NBLK)
_S_NP = np.where(_j < 93, _ROW_BLOCKS * (_j // 3) + (_j % 3), 0).astype(np.int32)
_k = np.arange(_DDP)
_TF_NP = np.where(_k < _DD, 48 * (_k % _D) + _k // _D, _ZROW).astype(np.int32)


def _sc_body(flat_ref, xs_ref, ys_ref, s_ref, tf_ref, out_ref,
             xv, yv, sv, tfv, idx0, idx1, st0, st1, ob0, ob1,
             gs0, gs1, os0, os1):
    wid = lax.axis_index("s") * 2 + lax.axis_index("c")
    pltpu.sync_copy(xs_ref.at[wid], xv)
    pltpu.sync_copy(ys_ref.at[wid], yv)
    pltpu.sync_copy(s_ref, sv)
    pltpu.sync_copy(tf_ref, tfv)

    iota16 = lax.iota(jnp.int32, 16)
    zero = jnp.zeros((16,), jnp.float32)

    def take(vec, idx):
        dnums = lax.GatherDimensionNumbers(
            offset_dims=(), collapsed_slice_dims=(0,), start_index_map=(0,))
        return lax.gather(vec, idx[:, None], dnums, slice_sizes=(1,),
                          mode=lax.GatherScatterMode.PROMISE_IN_BOUNDS)

    def splat(vec, lane):
        return take(vec, jnp.full((16,), 0, jnp.int32) + lane)

    def lane_sum(v):
        for off in (8, 4, 2, 1):
            v = v + take(v, iota16 ^ off)
        return v

    idx_refs = (idx0, idx1)
    stages = (st0, st1)
    obufs = (ob0, ob1)
    gsems = (gs0, gs1)
    osems = (os0, os1)

    for b in range(2):
        for s in range(_NP_):
            stages[b][s * _NBLK + 93, :] = zero

    def unit_coords(u):
        batch = u >> 3
        g = u & 7
        p1 = batch * 64 + g * _M
        out = []
        for h in range(2):
            pp = p1 + h * 256
            chunk = pp >> 4
            xc = xv[pl.ds(chunk * 16, 16)]
            yc = yv[pl.ds(chunk * 16, 16)]
            lane0 = pp & 15
            for r in range(_M):
                out.append((splat(xc, lane0 + r), splat(yc, lane0 + r)))
        return batch, out

    def fire(u, b):
        batch, coords = unit_coords(u)
        for s, (x, y) in enumerate(coords):
            job_base = (batch + 4 * (s // _M)) * _IMG_BLOCKS
            base = job_base + y * _ROW_BLOCKS + (x >> 4)
            for c in range(6):
                idx_refs[b][pl.ds(s * _NBLK + c * 16, 16)] = (
                    sv[pl.ds(c * 16, 16)] + base)
            pltpu.async_copy(
                flat_ref.at[idx_refs[b].at[pl.ds(s * _NBLK, 93)]],
                stages[b].at[pl.ds(s * _NBLK, 93)], gsems[b])

    def process(u, b, wait_out):
        stg = stages[b]
        ob = obufs[b]
        batch, coords = unit_coords(u)
        xms = [(x & 15) + s * (_NBLK * 16) for s, (x, _) in enumerate(coords)]
        for s in range(_NP_):
            pltpu.make_async_copy(
                flat_ref.at[idx_refs[b].at[pl.ds(s * _NBLK, 93)]],
                stg.at[pl.ds(s * _NBLK, 93)], gsems[b]).wait()
        if wait_out:
            pltpu.make_async_copy(ob, out_ref.at[pl.ds(0, _NP_)],
                                  osems[b]).wait()

        for h in range(2):
            rows = [jnp.full((16,), 0, jnp.int32) + (2 * r + h)
                    for r in range(_M)]

            def pass1(c, carry):
                accs, acc2s = carry
                tf = tfv[pl.ds(c * 16, 16)]
                col = iota16 + c * 16
                na, n2 = [], []
                for r in range(_M):
                    idx = tf + xms[h * _M + r]
                    g = plsc.load_gather(stg, [idx >> 4, idx & 15])
                    plsc.store_scatter(ob, [rows[r], col], g)
                    na.append(accs[r] + g)
                    n2.append(acc2s[r] + g * g)
                return tuple(na), tuple(n2)

            init = (tuple(zero for _ in range(_M)),
                    tuple(zero for _ in range(_M)))
            accs, acc2s = lax.fori_loop(0, 61, pass1, init)
            means, invs = [0] * _M, [0] * _M
            for r in range(_M):
                s1 = lane_sum(accs[r])
                s2 = lane_sum(acc2s[r])
                meanv = s1 * (1.0 / 961.0)
                varv = (s2 - s1 * meanv) * (1.0 / 960.0)
                varv = jnp.maximum(varv, 1e-30)
                rr = plsc.bitcast(
                    0x5F3759DF - (plsc.bitcast(varv, jnp.int32) >> 1),
                    jnp.float32)
                for _ in range(3):
                    rr = rr * (1.5 - 0.5 * varv * rr * rr)
                means[r] = meanv
                invs[r] = 1.0 / (varv * rr + 1e-4)

            def pass2(c, carry):
                for r in range(_M):
                    g = ob[2 * r + h, pl.ds(c * 16, 16)]
                    ob[2 * r + h, pl.ds(c * 16, 16)] = (
                        (g - means[r]) * invs[r])
                return carry

            lax.fori_loop(0, 61, pass2, 0)

        row0 = 2 * (batch * 2048 + wid * 64 + (u & 7) * _M)
        pltpu.async_copy(ob, out_ref.at[pl.ds(row0, _NP_)], osems[b])

    fire(jnp.int32(0), 0)
    fire(jnp.int32(1), 1)
    process(jnp.int32(0), 0, wait_out=False)
    fire(jnp.int32(2), 0)
    process(jnp.int32(1), 1, wait_out=False)
    fire(jnp.int32(3), 1)

    def main(i, carry):
        process(2 * i, 0, wait_out=True)
        fire(2 * i + 2, 0)
        process(2 * i + 1, 1, wait_out=True)
        fire(2 * i + 3, 1)
        return carry

    lax.fori_loop(1, _UNITS // 2 - 1, main, 0)
    process(jnp.int32(_UNITS - 2), 0, wait_out=True)
    process(jnp.int32(_UNITS - 1), 1, wait_out=True)
    pltpu.make_async_copy(ob0, out_ref.at[pl.ds(0, _NP_)], os0).wait()
    pltpu.make_async_copy(ob1, out_ref.at[pl.ds(0, _NP_)], os1).wait()


@jax.jit
def kernel(image1, image2, matches):
    imgs = jnp.concatenate([image1[:, 0], image2[:, 0]], axis=0)
    padded = jnp.pad(imgs, ((0, 0), (15, 15), (15, 17)))
    flat = padded.reshape(_NJOBS * _IMG_BLOCKS, 16)

    xs = jnp.concatenate([matches[..., 0], matches[..., 2]], axis=0)
    ys = jnp.concatenate([matches[..., 1], matches[..., 3]], axis=0)
    xs_t = xs.reshape(_NJOBS, _NW, 64).transpose(1, 0, 2).reshape(_NW, 512)
    ys_t = ys.reshape(_NJOBS, _NW, 64).transpose(1, 0, 2).reshape(_NW, 512)

    mesh = plsc.VectorSubcoreMesh(core_axis_name="c", subcore_axis_name="s",
                                  num_cores=2, num_subcores=16)
    run = pl.kernel(
        _sc_body,
        out_type=jax.ShapeDtypeStruct((2 * 4 * 2048, _DDP), jnp.float32),
        mesh=mesh,
        compiler_params=pltpu.CompilerParams(needs_layout_passes=False,
                                             use_tc_tiling_on_sc=False),
        scratch_types=[
            pltpu.VMEM((512,), jnp.int32),
            pltpu.VMEM((512,), jnp.int32),
            pltpu.VMEM((_NBLK,), jnp.int32),
            pltpu.VMEM((_DDP,), jnp.int32),
            pltpu.VMEM((_NP_ * _NBLK,), jnp.int32),
            pltpu.VMEM((_NP_ * _NBLK,), jnp.int32),
            pltpu.VMEM((_NP_ * _NBLK, 16), jnp.float32),
            pltpu.VMEM((_NP_ * _NBLK, 16), jnp.float32),
            pltpu.VMEM((_NP_, _DDP), jnp.float32),
            pltpu.VMEM((_NP_, _DDP), jnp.float32),
            pltpu.SemaphoreType.DMA,
            pltpu.SemaphoreType.DMA,
            pltpu.SemaphoreType.DMA,
            pltpu.SemaphoreType.DMA,
        ],
    )
    out = run(flat, xs_t, ys_t, jnp.asarray(_S_NP), jnp.asarray(_TF_NP))
    out = out.reshape(4, 2048, 2, _DDP)
    return jnp.concatenate([out[:, :, 0, :_DD], out[:, :, 1, :_DD]], axis=-1)


# speedup vs baseline: 1.5195x; 1.5195x over previous
import numpy as np
import jax
import jax.numpy as jnp
from jax import lax
from jax.experimental import pallas as pl
from jax.experimental.pallas import tpu as pltpu
from jax.experimental.pallas import tpu_sc as plsc

_D = 31
_DD = _D * _D
_DDP = 976
_ROW_BLOCKS = 34
_IMG_BLOCKS = 542 * _ROW_BLOCKS
_NJOBS = 8
_NW = 32
_M = 8
_UNITS = 4 * 2048 // (_NW * _M)
_NBLK = 96
_NP_ = 2 * _M
_ZROW = 93 * 16

_j = np.arange(_NBLK)
_S_NP = np.where(_j < 93, _ROW_BLOCKS * (_j // 3) + (_j % 3), 0).astype(np.int32)
_k = np.arange(_DDP)
_TF_NP = np.where(_k < _DD, 48 * (_k % _D) + _k // _D, _ZROW).astype(np.int32)
_TK_NP = np.where(_k < _DD, _k, 0).astype(np.int32)


def _sc_body(flat_ref, xs_ref, ys_ref, s_ref, tf_ref, tk_ref, out_ref,
             xv, yv, sv, tfv, tkv, idx0, idx1, st0, st1, ob0, ob1,
             gs0, gs1, os0, os1):
    wid = lax.axis_index("s") * 2 + lax.axis_index("c")
    pltpu.sync_copy(xs_ref.at[wid], xv)
    pltpu.sync_copy(ys_ref.at[wid], yv)
    pltpu.sync_copy(s_ref, sv)
    pltpu.sync_copy(tf_ref, tfv)
    pltpu.sync_copy(tk_ref, tkv)

    iota16 = lax.iota(jnp.int32, 16)
    zero = jnp.zeros((16,), jnp.float32)

    def take(vec, idx):
        dnums = lax.GatherDimensionNumbers(
            offset_dims=(), collapsed_slice_dims=(0,), start_index_map=(0,))
        return lax.gather(vec, idx[:, None], dnums, slice_sizes=(1,),
                          mode=lax.GatherScatterMode.PROMISE_IN_BOUNDS)

    def splat(vec, lane):
        return take(vec, jnp.full((16,), 0, jnp.int32) + lane)

    def lane_sum(v):
        for off in (8, 4, 2, 1):
            v = v + take(v, iota16 ^ off)
        return v

    idx_refs = (idx0, idx1)
    stages = (st0, st1)
    obufs = (ob0, ob1)
    gsems = (gs0, gs1)
    osems = (os0, os1)

    tail_mask = iota16 < 1

    def unit_coords(u):
        batch = u >> 3
        g = u & 7
        p1 = batch * 64 + g * _M
        out = []
        for h in range(2):
            pp = p1 + h * 256
            chunk = pp >> 4
            xc = xv[pl.ds(chunk * 16, 16)]
            yc = yv[pl.ds(chunk * 16, 16)]
            lane0 = pp & 15
            for r in range(_M):
                out.append((splat(xc, lane0 + r), splat(yc, lane0 + r)))
        return batch, out

    def fire(u, b):
        batch, coords = unit_coords(u)
        for s, (x, y) in enumerate(coords):
            job_base = (batch + 4 * (s // _M)) * _IMG_BLOCKS
            base = job_base + y * _ROW_BLOCKS + (x >> 4)
            for c in range(6):
                idx_refs[b][pl.ds(s * _NBLK + c * 16, 16)] = (
                    sv[pl.ds(c * 16, 16)] + base)
        pltpu.async_copy(
            flat_ref.at[idx_refs[b].at[pl.ds(0, _NP_ * _NBLK)]],
            stages[b].at[pl.ds(0, _NP_ * _NBLK)], gsems[b])

    def process(u, b, wait_out):
        stg = stages[b]
        ob = obufs[b]
        batch, coords = unit_coords(u)
        xms = [(x & 15) + s * (_NBLK * 16) for s, (x, _) in enumerate(coords)]
        pltpu.make_async_copy(
            flat_ref.at[idx_refs[b].at[pl.ds(0, _NP_ * _NBLK)]],
            stg.at[pl.ds(0, _NP_ * _NBLK)], gsems[b]).wait()

        means, invs = [0] * _NP_, [0] * _NP_
        for h in range(2):
            def pass1(c, carry):
                accs, acc2s = carry
                tf = tfv[pl.ds(c * 16, 16)]
                na, n2 = [], []
                for r in range(_M):
                    idx = tf + xms[h * _M + r]
                    g = plsc.load_gather(stg, [idx >> 4, idx & 15])
                    na.append(accs[r] + g)
                    n2.append(acc2s[r] + g * g)
                return tuple(na), tuple(n2)

            init = (tuple(zero for _ in range(_M)),
                    tuple(zero for _ in range(_M)))
            accs, acc2s = lax.fori_loop(0, 60, pass1, init)
            accs, acc2s = list(accs), list(acc2s)
            tf60 = tfv[pl.ds(960, 16)]
            for r in range(_M):
                idx = tf60 + xms[h * _M + r]
                g = plsc.load_gather(stg, [idx >> 4, idx & 15])
                gm = jnp.where(tail_mask, g, 0.0)
                accs[r] = accs[r] + gm
                acc2s[r] = acc2s[r] + gm * gm
            for r in range(_M):
                s1 = lane_sum(accs[r])
                s2 = lane_sum(acc2s[r])
                meanv = s1 * (1.0 / 961.0)
                varv = (s2 - s1 * meanv) * (1.0 / 960.0)
                varv = jnp.maximum(varv, 1e-30)
                rr = plsc.bitcast(
                    0x5F3759DF - (plsc.bitcast(varv, jnp.int32) >> 1),
                    jnp.float32)
                for _ in range(3):
                    rr = rr * (1.5 - 0.5 * varv * rr * rr)
                means[h * _M + r] = meanv
                invs[h * _M + r] = 1.0 / (varv * rr + 1e-4)

        if wait_out:
            pltpu.make_async_copy(ob, out_ref.at[pl.ds(0, _M)],
                                  osems[b]).wait()

        for h in range(2):
            def pass2(c, carry):
                tf = tfv[pl.ds(c * 16, 16)]
                col = tkv[pl.ds(c * 16, 16)] + (h * _DD)
                for r in range(_M):
                    s = h * _M + r
                    idx = tf + xms[s]
                    g = plsc.load_gather(stg, [idx >> 4, idx & 15])
                    val = (g - means[s]) * invs[s]
                    rowv = jnp.full((16,), 0, jnp.int32) + r
                    plsc.store_scatter(ob, [rowv, col], val)
                return carry

            lax.fori_loop(0, 60, pass2, 0)
            tf60 = tfv[pl.ds(960, 16)]
            col60 = tkv[pl.ds(960, 16)] + (h * _DD)
            for r in range(_M):
                s = h * _M + r
                idx = tf60 + xms[s]
                g = plsc.load_gather(stg, [idx >> 4, idx & 15])
                val = (g - means[s]) * invs[s]
                rowv = jnp.full((16,), 0, jnp.int32) + r
                plsc.store_scatter(ob, [rowv, col60], val, mask=tail_mask)

        row0 = batch * 2048 + wid * 64 + (u & 7) * _M
        pltpu.async_copy(ob, out_ref.at[pl.ds(row0, _M)], osems[b])

    fire(jnp.int32(0), 0)
    fire(jnp.int32(1), 1)
    process(jnp.int32(0), 0, wait_out=False)
    fire(jnp.int32(2), 0)
    process(jnp.int32(1), 1, wait_out=False)
    fire(jnp.int32(3), 1)

    def main(i, carry):
        process(2 * i, 0, wait_out=True)
        fire(2 * i + 2, 0)
        process(2 * i + 1, 1, wait_out=True)
        fire(2 * i + 3, 1)
        return carry

    lax.fori_loop(1, _UNITS // 2 - 1, main, 0)
    process(jnp.int32(_UNITS - 2), 0, wait_out=True)
    process(jnp.int32(_UNITS - 1), 1, wait_out=True)
    pltpu.make_async_copy(ob0, out_ref.at[pl.ds(0, _M)], os0).wait()
    pltpu.make_async_copy(ob1, out_ref.at[pl.ds(0, _M)], os1).wait()


@jax.jit
def kernel(image1, image2, matches):
    imgs = jnp.concatenate([image1[:, 0], image2[:, 0]], axis=0)
    padded = jnp.pad(imgs, ((0, 0), (15, 15), (15, 17)))
    flat = padded.reshape(_NJOBS * _IMG_BLOCKS, 16)

    xs = jnp.concatenate([matches[..., 0], matches[..., 2]], axis=0)
    ys = jnp.concatenate([matches[..., 1], matches[..., 3]], axis=0)
    xs_t = xs.reshape(_NJOBS, _NW, 64).transpose(1, 0, 2).reshape(_NW, 512)
    ys_t = ys.reshape(_NJOBS, _NW, 64).transpose(1, 0, 2).reshape(_NW, 512)

    mesh = plsc.VectorSubcoreMesh(core_axis_name="c", subcore_axis_name="s",
                                  num_cores=2, num_subcores=16)
    run = pl.kernel(
        _sc_body,
        out_type=jax.ShapeDtypeStruct((4 * 2048, 2 * _DD), jnp.float32),
        mesh=mesh,
        compiler_params=pltpu.CompilerParams(needs_layout_passes=False,
                                             use_tc_tiling_on_sc=False),
        scratch_types=[
            pltpu.VMEM((512,), jnp.int32),
            pltpu.VMEM((512,), jnp.int32),
            pltpu.VMEM((_NBLK,), jnp.int32),
            pltpu.VMEM((_DDP,), jnp.int32),
            pltpu.VMEM((_DDP,), jnp.int32),
            pltpu.VMEM((_NP_ * _NBLK,), jnp.int32),
            pltpu.VMEM((_NP_ * _NBLK,), jnp.int32),
            pltpu.VMEM((_NP_ * _NBLK, 16), jnp.float32),
            pltpu.VMEM((_NP_ * _NBLK, 16), jnp.float32),
            pltpu.VMEM((_M, 2 * _DD), jnp.float32),
            pltpu.VMEM((_M, 2 * _DD), jnp.float32),
            pltpu.SemaphoreType.DMA,
            pltpu.SemaphoreType.DMA,
            pltpu.SemaphoreType.DMA,
            pltpu.SemaphoreType.DMA,
        ],
    )
    out = run(flat, xs_t, ys_t, jnp.asarray(_S_NP), jnp.asarray(_TF_NP),
              jnp.asarray(_TK_NP))
    return out.reshape(4, 2048, 2 * _DD)
